# baseline (device time: 40797 ns/iter reference)
import jax
import jax.numpy as jnp
from jax import lax
from jax.experimental import pallas as pl
from jax.experimental.pallas import tpu as pltpu

N_DEV = 8
M_PER = 128
K = 1024
N_PER = 128


def _gelu(y):
    c = 0.7978845608028654
    return 0.5 * y * (1.0 + jnp.tanh(c * (y + 0.044715 * y * y * y)))


def kernel(x, w_mat):
    def body(x_ref, w_ref, out_ref, xg_ref, send_sems, recv_sems):
        my = lax.axis_index("i")
        left = lax.rem(my - 1 + N_DEV, N_DEV)
        right = lax.rem(my + 1, N_DEV)

        barrier_sem = pltpu.get_barrier_semaphore()
        for nbr in (left, right):
            pl.semaphore_signal(
                barrier_sem, inc=1,
                device_id=(nbr,), device_id_type=pl.DeviceIdType.MESH,
            )
        pl.semaphore_wait(barrier_sem, 2)

        xg_ref[0] = x_ref[:, :].astype(jnp.bfloat16)
        w16 = w_ref[:, :].astype(jnp.bfloat16)

        def compute_slot(j):
            o = lax.rem(my - j + N_DEV, N_DEV)
            y = jnp.dot(xg_ref[j], w16, preferred_element_type=jnp.float32)
            out_ref[pl.ds(o * M_PER, M_PER), :] = _gelu(y)

        compute_slot(0)

        for h in range(N_DEV - 1):
            rdma = pltpu.make_async_remote_copy(
                src_ref=xg_ref.at[h],
                dst_ref=xg_ref.at[h + 1],
                send_sem=send_sems.at[h],
                recv_sem=recv_sems.at[h],
                device_id=(right,),
                device_id_type=pl.DeviceIdType.MESH,
            )
            rdma.start()
            rdma.wait()
            compute_slot(h + 1)

    return pl.pallas_call(
        body,
        out_shape=jax.ShapeDtypeStruct((N_DEV * M_PER, N_PER), jnp.float32),
        in_specs=[
            pl.BlockSpec(memory_space=pltpu.VMEM),
            pl.BlockSpec(memory_space=pltpu.VMEM),
        ],
        out_specs=pl.BlockSpec(memory_space=pltpu.VMEM),
        scratch_shapes=[
            pltpu.VMEM((N_DEV, M_PER, K), jnp.bfloat16),
            pltpu.SemaphoreType.DMA((N_DEV - 1,)),
            pltpu.SemaphoreType.DMA((N_DEV - 1,)),
        ],
        compiler_params=pltpu.CompilerParams(collective_id=0),
    )(x, w_mat)


# device time: 23747 ns/iter; 1.7180x vs baseline; 1.7180x over previous
import jax
import jax.numpy as jnp
from jax import lax
from jax.experimental import pallas as pl
from jax.experimental.pallas import tpu as pltpu

N_DEV = 8
M_PER = 128
K = 1024
N_PER = 128


def _gelu(y):
    c = 0.7978845608028654
    return 0.5 * y * (1.0 + jnp.tanh(c * (y + 0.044715 * y * y * y)))


def kernel(x, w_mat):
    def body(x_ref, w_ref, out_ref, own_ref, recv_ref, send_sems, recv_sems):
        my = lax.axis_index("i")

        barrier_sem = pltpu.get_barrier_semaphore()
        for d in range(1, N_DEV):
            t = lax.rem(my + d, N_DEV)
            pl.semaphore_signal(
                barrier_sem, inc=1,
                device_id=(t,), device_id_type=pl.DeviceIdType.MESH,
            )
        pl.semaphore_wait(barrier_sem, N_DEV - 1)

        own_ref[:, :] = x_ref[:, :].astype(jnp.bfloat16)

        rdmas = []
        for d in range(1, N_DEV):
            t = lax.rem(my + d, N_DEV)
            rdma = pltpu.make_async_remote_copy(
                src_ref=own_ref,
                dst_ref=recv_ref.at[d - 1],
                send_sem=send_sems.at[d - 1],
                recv_sem=recv_sems.at[d - 1],
                device_id=(t,),
                device_id_type=pl.DeviceIdType.MESH,
            )
            rdma.start()
            rdmas.append(rdma)

        w16 = w_ref[:, :].astype(jnp.bfloat16)

        y = jnp.dot(own_ref[:, :], w16, preferred_element_type=jnp.float32)
        out_ref[pl.ds(my * M_PER, M_PER), :] = _gelu(y)

        for d in range(1, N_DEV):
            rdmas[d - 1].wait_recv()
            o = lax.rem(my - d + N_DEV, N_DEV)
            y = jnp.dot(recv_ref[d - 1], w16, preferred_element_type=jnp.float32)
            out_ref[pl.ds(o * M_PER, M_PER), :] = _gelu(y)

        for rdma in rdmas:
            rdma.wait_send()

    return pl.pallas_call(
        body,
        out_shape=jax.ShapeDtypeStruct((N_DEV * M_PER, N_PER), jnp.float32),
        in_specs=[
            pl.BlockSpec(memory_space=pltpu.VMEM),
            pl.BlockSpec(memory_space=pltpu.VMEM),
        ],
        out_specs=pl.BlockSpec(memory_space=pltpu.VMEM),
        scratch_shapes=[
            pltpu.VMEM((M_PER, K), jnp.bfloat16),
            pltpu.VMEM((N_DEV - 1, M_PER, K), jnp.bfloat16),
            pltpu.SemaphoreType.DMA((N_DEV - 1,)),
            pltpu.SemaphoreType.DMA((N_DEV - 1,)),
        ],
        compiler_params=pltpu.CompilerParams(collective_id=0),
    )(x, w_mat)


# device time: 18914 ns/iter; 2.1570x vs baseline; 1.2555x over previous
import jax
import jax.numpy as jnp
from jax import lax
from jax.experimental import pallas as pl
from jax.experimental.pallas import tpu as pltpu

N_DEV = 8
M_PER = 128
K = 1024
N_PER = 128


def _gelu(y):
    c = 0.7978845608028654
    return 0.5 * y * (1.0 + jnp.tanh(c * (y + 0.044715 * y * y * y)))


def kernel(x, w_mat):
    def body(x_ref, w_ref, out_ref, own_ref, recv_ref, send_sems, recv_sems):
        my = lax.axis_index("i")

        z = my // 4
        p = my % 4
        y = p // 2
        xc = jnp.logical_or(p == 1, p == 2).astype(my.dtype)

        def pos(px, py, pz):
            return 4 * pz + 2 * py + jnp.bitwise_xor(px, py)

        xn = pos(1 - xc, y, z)
        yn = pos(xc, 1 - y, z)
        zn = pos(xc, y, 1 - z)
        dg = pos(1 - xc, 1 - y, 1 - z)

        slot_origin = [xn, yn, zn, dg,
                       pos(1 - xc, 1 - y, z),
                       pos(xc, 1 - y, 1 - z),
                       pos(1 - xc, y, 1 - z)]

        barrier_sem = pltpu.get_barrier_semaphore()
        for t in (xn, yn, zn, dg):
            pl.semaphore_signal(
                barrier_sem, inc=1,
                device_id=(t,), device_id_type=pl.DeviceIdType.MESH,
            )
        pl.semaphore_wait(barrier_sem, 4)

        own_ref[:, :] = x_ref[:, :].astype(jnp.bfloat16)

        def copy(src, dst_slot, sem_id, target):
            return pltpu.make_async_remote_copy(
                src_ref=src,
                dst_ref=recv_ref.at[dst_slot],
                send_sem=send_sems.at[sem_id],
                recv_sem=recv_sems.at[dst_slot],
                device_id=(target,),
                device_id_type=pl.DeviceIdType.MESH,
            )

        sends = [
            copy(own_ref, 0, 0, xn),
            copy(own_ref, 1, 1, yn),
            copy(own_ref, 2, 2, zn),
            copy(own_ref, 3, 3, dg),
        ]
        for s in sends:
            s.start()

        w16 = w_ref[:, :].astype(jnp.bfloat16)

        def compute(src, origin_pos):
            yy = jnp.dot(src, w16, preferred_element_type=jnp.float32)
            out_ref[pl.ds(origin_pos * M_PER, M_PER), :] = _gelu(yy)

        compute(own_ref[:, :], my)

        recvs = [copy(own_ref, j, 0, my) for j in range(N_DEV - 1)]

        recvs[1].wait_recv()
        fwd_xy = copy(recv_ref.at[1], 4, 4, xn)
        fwd_xy.start()
        compute(recv_ref[1], slot_origin[1])

        recvs[2].wait_recv()
        fwd_yz = copy(recv_ref.at[2], 5, 5, yn)
        fwd_yz.start()
        compute(recv_ref[2], slot_origin[2])

        recvs[0].wait_recv()
        fwd_xz = copy(recv_ref.at[0], 6, 6, zn)
        fwd_xz.start()
        compute(recv_ref[0], slot_origin[0])

        for j in (3, 4, 5, 6):
            recvs[j].wait_recv()
            compute(recv_ref[j], slot_origin[j])

        for s in sends + [fwd_xy, fwd_yz, fwd_xz]:
            s.wait_send()

    return pl.pallas_call(
        body,
        out_shape=jax.ShapeDtypeStruct((N_DEV * M_PER, N_PER), jnp.float32),
        in_specs=[
            pl.BlockSpec(memory_space=pltpu.VMEM),
            pl.BlockSpec(memory_space=pltpu.VMEM),
        ],
        out_specs=pl.BlockSpec(memory_space=pltpu.VMEM),
        scratch_shapes=[
            pltpu.VMEM((M_PER, K), jnp.bfloat16),
            pltpu.VMEM((N_DEV - 1, M_PER, K), jnp.bfloat16),
            pltpu.SemaphoreType.DMA((N_DEV - 1,)),
            pltpu.SemaphoreType.DMA((N_DEV - 1,)),
        ],
        compiler_params=pltpu.CompilerParams(collective_id=0),
    )(x, w_mat)


# device time: 17425 ns/iter; 2.3413x vs baseline; 1.0855x over previous
import jax
import jax.numpy as jnp
from jax import lax
from jax.experimental import pallas as pl
from jax.experimental.pallas import tpu as pltpu

N_DEV = 8
M_PER = 128
H = 64
K = 1024
N_PER = 128

SX, SY, SZ, SBD, SXY, SYZ, SXZ = range(7)


def _gelu(y):
    c = 0.7978845608028654
    return 0.5 * y * (1.0 + jnp.tanh(c * (y + 0.044715 * y * y * y)))


def kernel(x, w_mat):
    def body(x_ref, w_ref, out_ref, own_ref, recv_ref, send_sems, recv_sems):
        my = lax.axis_index("i")

        z = my // 4
        p = my % 4
        y = p // 2
        xc = jnp.logical_or(p == 1, p == 2).astype(my.dtype)

        def pos(px, py, pz):
            return 4 * pz + 2 * py + jnp.bitwise_xor(px, py)

        xn = pos(1 - xc, y, z)
        yn = pos(xc, 1 - y, z)
        zn = pos(xc, y, 1 - z)

        slot_origin = {
            SX: xn, SY: yn, SZ: zn,
            SBD: pos(1 - xc, 1 - y, 1 - z),
            SXY: pos(1 - xc, 1 - y, z),
            SYZ: pos(xc, 1 - y, 1 - z),
            SXZ: pos(1 - xc, y, 1 - z),
        }

        barrier_sem = pltpu.get_barrier_semaphore()
        for t in (xn, yn, zn):
            pl.semaphore_signal(
                barrier_sem, inc=1,
                device_id=(t,), device_id_type=pl.DeviceIdType.MESH,
            )
        pl.semaphore_wait(barrier_sem, 3)

        own_ref[:, :] = x_ref[:, :].astype(jnp.bfloat16)

        A = pl.ds(0, H)
        B = pl.ds(H, H)

        def copy(src, dst_slot, half, sem_id, target):
            return pltpu.make_async_remote_copy(
                src_ref=src,
                dst_ref=recv_ref.at[dst_slot, half],
                send_sem=send_sems.at[sem_id],
                recv_sem=recv_sems.at[sem_id],
                device_id=(target,),
                device_id_type=pl.DeviceIdType.MESH,
            )

        p1 = [
            copy(own_ref.at[A], SX, A, 0, xn),
            copy(own_ref.at[A], SY, A, 1, yn),
            copy(own_ref.at[A], SZ, A, 2, zn),
            copy(own_ref.at[B], SX, B, 3, xn),
            copy(own_ref.at[B], SY, B, 4, yn),
            copy(own_ref.at[B], SZ, B, 5, zn),
        ]
        for s in p1:
            s.start()

        w16 = w_ref[:, :].astype(jnp.bfloat16)

        def compute(src, origin_pos):
            yy = jnp.dot(src, w16, preferred_element_type=jnp.float32)
            out_ref[pl.ds(origin_pos * M_PER, M_PER), :] = _gelu(yy)

        compute(own_ref[:, :], my)

        sem_dst = {
            0: (SX, A), 1: (SY, A), 2: (SZ, A),
            3: (SX, B), 4: (SY, B), 5: (SZ, B),
            6: (SXY, A), 7: (SXY, B), 8: (SYZ, A), 9: (SYZ, B),
            10: (SXZ, A), 11: (SXZ, B), 12: (SBD, A), 13: (SBD, B),
        }

        def wait(sem_id):
            slot, half = sem_dst[sem_id]
            pltpu.make_async_remote_copy(
                src_ref=recv_ref.at[slot, half],
                dst_ref=recv_ref.at[slot, half],
                send_sem=send_sems.at[sem_id],
                recv_sem=recv_sems.at[sem_id],
                device_id=(my,),
                device_id_type=pl.DeviceIdType.MESH,
            ).wait_recv()

        p2 = []
        for sem_in, rdma_args in (
            (0, (recv_ref.at[SX, A], SXZ, A, 10, zn)),
            (1, (recv_ref.at[SY, A], SXY, A, 6, xn)),
            (2, (recv_ref.at[SZ, A], SYZ, A, 8, yn)),
            (3, (recv_ref.at[SX, B], SXY, B, 7, yn)),
            (4, (recv_ref.at[SY, B], SYZ, B, 9, zn)),
            (5, (recv_ref.at[SZ, B], SXZ, B, 11, xn)),
        ):
            wait(sem_in)
            r = copy(*rdma_args)
            r.start()
            p2.append(r)

        compute(recv_ref[SX], slot_origin[SX])
        compute(recv_ref[SY], slot_origin[SY])
        compute(recv_ref[SZ], slot_origin[SZ])

        wait(8)
        p3a = copy(recv_ref.at[SYZ, A], SBD, A, 12, xn)
        p3a.start()
        wait(11)
        p3b = copy(recv_ref.at[SXZ, B], SBD, B, 13, yn)
        p3b.start()

        wait(6)
        wait(7)
        compute(recv_ref[SXY], slot_origin[SXY])
        wait(9)
        compute(recv_ref[SYZ], slot_origin[SYZ])
        wait(10)
        compute(recv_ref[SXZ], slot_origin[SXZ])

        wait(12)
        wait(13)
        compute(recv_ref[SBD], slot_origin[SBD])

        for s in p1 + p2 + [p3a, p3b]:
            s.wait_send()

    return pl.pallas_call(
        body,
        out_shape=jax.ShapeDtypeStruct((N_DEV * M_PER, N_PER), jnp.float32),
        in_specs=[
            pl.BlockSpec(memory_space=pltpu.VMEM),
            pl.BlockSpec(memory_space=pltpu.VMEM),
        ],
        out_specs=pl.BlockSpec(memory_space=pltpu.VMEM),
        scratch_shapes=[
            pltpu.VMEM((M_PER, K), jnp.bfloat16),
            pltpu.VMEM((N_DEV - 1, M_PER, K), jnp.bfloat16),
            pltpu.SemaphoreType.DMA((14,)),
            pltpu.SemaphoreType.DMA((14,)),
        ],
        compiler_params=pltpu.CompilerParams(collective_id=0),
    )(x, w_mat)
